# baseline (speedup 1.0000x reference)
import functools

import jax
import jax.numpy as jnp
from jax import lax
from jax.experimental import pallas as pl
from jax.experimental.pallas import tpu as pltpu
from jax.experimental.pallas import tpu_sc as plsc

_NUM_POINTS = 512


@functools.lru_cache(maxsize=None)
def _make_sc_gather(V, D, B):
    info = plsc.get_sparse_core_info()
    NC, NS = info.num_cores, info.num_subcores
    NW = NC * NS
    assert D % info.num_lanes == 0 and B % (8 * NW) == 0
    b_per_w = B // NW
    mesh = plsc.VectorSubcoreMesh(core_axis_name="c", subcore_axis_name="s")

    @functools.partial(
        pl.kernel,
        mesh=mesh,
        out_type=jax.ShapeDtypeStruct((B, D), jnp.float32),
        scratch_types=[
            pltpu.VMEM((b_per_w,), jnp.int32),
            pltpu.VMEM((b_per_w, D), jnp.float32),
            pltpu.SemaphoreType.DMA,
        ],
    )
    def k(table_hbm, idx_hbm, out_hbm, idx_v, rows_v, sem):
        wid = lax.axis_index("s") * NC + lax.axis_index("c")
        base = wid * b_per_w
        pltpu.sync_copy(idx_hbm.at[pl.ds(base, b_per_w)], idx_v)
        pltpu.async_copy(table_hbm.at[idx_v], rows_v, sem).wait()
        pltpu.sync_copy(rows_v, out_hbm.at[pl.ds(base, b_per_w)])

    return k


def kernel(features):
    B = features.shape[0]
    C = features.shape[-1]
    pts = features.reshape(B, -1, C)
    N = pts.shape[1]
    centroid = jnp.mean(pts, axis=1, keepdims=True)
    distances = jnp.linalg.norm(pts - centroid, axis=2)
    sorted_indices = jnp.argsort(distances, axis=1)
    step = N // _NUM_POINTS
    if step == 0:
        step = 1
    sampled_indices = sorted_indices[:, ::step][:, :_NUM_POINTS]

    flat_idx = (sampled_indices.astype(jnp.int32)
                + (jnp.arange(B, dtype=jnp.int32) * N)[:, None]).reshape(-1)
    table = pts.reshape(B * N // 2, 2 * C)
    rows = _make_sc_gather(B * N // 2, 2 * C, B * _NUM_POINTS)(
        table, flat_idx >> 1)
    gathered = jnp.where((flat_idx & 1)[:, None] == 0, rows[:, :C], rows[:, C:])
    sampled_points = gathered.reshape(B, _NUM_POINTS, C)
    return (sampled_points, sampled_indices)


# speedup vs baseline: 1.6767x; 1.6767x over previous
import functools

import jax
import jax.numpy as jnp
from jax import lax
from jax.experimental import pallas as pl
from jax.experimental.pallas import tpu as pltpu
from jax.experimental.pallas import tpu_sc as plsc

_NUM_POINTS = 512
_NBIN = 1 << 14
_CAP = 16384
_CH = 4096
_NSLOT = 512


def _iota16():
    return lax.broadcasted_iota(jnp.int32, (16,), 0)


def _full16(s):
    return jnp.full((16,), s, jnp.int32)


def _srl(x, n):
    return lax.shift_right_logical(x, n if isinstance(n, jax.Array) else _full16(n))


def _sll(x, n):
    return lax.shift_left(x, n if isinstance(n, jax.Array) else _full16(n))


def _first_target(base, size):
    ft = _sll(_srl(base + 127, 7), 7)
    return ft < base + size, ft


@functools.lru_cache(maxsize=None)
def _make_select(B, N):
    mesh = plsc.VectorSubcoreMesh(core_axis_name="c", subcore_axis_name="s")
    info = plsc.get_sparse_core_info()
    NC = info.num_cores
    nch = N // _CH

    @functools.partial(
        pl.kernel,
        mesh=mesh,
        out_type=jax.ShapeDtypeStruct((B, _NUM_POINTS), jnp.int32),
        compiler_params=pltpu.CompilerParams(needs_layout_passes=False),
        scratch_types=[
            pltpu.VMEM((_CH,), jnp.int32),
            pltpu.VMEM((2, 16), jnp.int32),
            pltpu.VMEM((_NBIN,), jnp.int32),
            pltpu.VMEM((_NBIN,), jnp.int32),
            pltpu.VMEM((_NBIN,), jnp.int32),
            pltpu.VMEM((_CAP,), jnp.int32),
            pltpu.VMEM((_CAP,), jnp.int32),
            pltpu.VMEM((_CAP,), jnp.int32),
            pltpu.VMEM((_NSLOT,), jnp.int32),
            pltpu.VMEM((_NSLOT,), jnp.int32),
            pltpu.VMEM((_NUM_POINTS,), jnp.int32),
            pltpu.SemaphoreType.DMA,
        ],
    )
    def k(u_hbm, prm_hbm, out_hbm, ubuf, pbuf, hist, cum, smap,
          vcand, icand, scand, sb_a, sb_b, outb, sem):
        del sem
        wid = lax.axis_index("s") * NC + lax.axis_index("c")

        @pl.when(wid < B)
        def _body():
            row = wid
            pltpu.sync_copy(prm_hbm.at[row], pbuf)
            umin_v = pbuf[0, :]
            shift_v = pbuf[1, :]
            zeros = jnp.zeros((16,), jnp.int32)

            def _z(i, _):
                hist[pl.ds(i * 16, 16)] = zeros
                return 0
            lax.fori_loop(0, _NBIN // 16, _z, 0)

            def _zo(i, _):
                outb[pl.ds(i * 16, 16)] = zeros
                return 0
            lax.fori_loop(0, _NUM_POINTS // 16, _zo, 0)

            for c in range(nch):
                pltpu.sync_copy(u_hbm.at[row, pl.ds(c * _CH, _CH)], ubuf)

                def _ha(j, _):
                    uv = ubuf[pl.ds(j * 16, 16)]
                    v = _srl(uv - umin_v, shift_v)
                    cnt, lastm = plsc.scan_count(v)
                    plsc.addupdate_scatter(hist, [v], cnt, mask=lastm)
                    return 0
                lax.fori_loop(0, _CH // 16, _ha, 0)

            def _pb(i, carry):
                h = hist[pl.ds(i * 16, 16)]
                s = plsc.cumsum(h)
                cum[pl.ds(i * 16, 16)] = s - h + _full16(carry)
                return carry + jnp.sum(h)
            lax.fori_loop(0, _NBIN // 16, _pb, jnp.int32(0))

            def _b2(i, nsl):
                sz = hist[pl.ds(i * 16, 16)]
                base = cum[pl.ds(i * 16, 16)]
                hast, _ = _first_target(base, sz)
                act = (sz > 1) & hast
                ai = jnp.where(act, 1, 0).astype(jnp.int32)
                pre = plsc.cumsum(ai) - ai + _full16(nsl)
                smap[pl.ds(i * 16, 16)] = pre
                plsc.store_scatter(sb_a, [jnp.where(act, pre, 0)], base,
                                   mask=act)
                return nsl + jnp.sum(ai)
            nslots = lax.fori_loop(0, _NBIN // 16, _b2, jnp.int32(0))

            ncand = jnp.int32(0)
            for c in range(nch):
                pltpu.sync_copy(u_hbm.at[row, pl.ds(c * _CH, _CH)], ubuf)

                def _pc(j, off, _c=c):
                    uv = ubuf[pl.ds(j * 16, 16)]
                    vfull = uv - umin_v
                    v = _srl(vfull, shift_v)
                    b0 = plsc.load_gather(cum, [v])
                    sz = plsc.load_gather(hist, [v])
                    hast, _ = _first_target(b0, sz)
                    idxv = _iota16() + _full16(_c * _CH + j * 16)
                    res1 = hast & (sz == 1)
                    plsc.store_scatter(outb, [_srl(b0, 7)], idxv, mask=res1)
                    cand = hast & (sz > 1)
                    sl = plsc.load_gather(smap, [v])
                    offc = jnp.minimum(off, _CAP - 16)
                    cm = jnp.where(cand, 1, 0).astype(jnp.int32)
                    plsc.store_compressed(vcand.at[pl.ds(offc, 16)], vfull, mask=cand)
                    plsc.store_compressed(icand.at[pl.ds(offc, 16)], idxv, mask=cand)
                    plsc.store_compressed(scand.at[pl.ds(offc, 16)], sl, mask=cand)
                    return off + jnp.sum(cm)
                ncand = lax.fori_loop(0, _CH // 16, _pc, ncand)

            sb_cur, sb_new = sb_a, sb_b
            for t in range(4):
                sh_v = jnp.maximum(shift_v - 5 * (t + 1), 0)
                tb = nslots * 2
                tc = (ncand + 15) // 16

                def _zr(i, _):
                    hist[pl.ds(i * 16, 16)] = zeros
                    return 0
                lax.fori_loop(0, tb, _zr, 0)

                def _hc(i, _):
                    valid = _iota16() + _full16(i * 16) < _full16(ncand)
                    vv = vcand[pl.ds(i * 16, 16)]
                    sv = scand[pl.ds(i * 16, 16)]
                    dig = _srl(vv, sh_v) & 31
                    addr = jnp.where(valid, _sll(sv, 5) + dig, 0)
                    cnt, lastm = plsc.scan_count(addr, mask=valid)
                    plsc.addupdate_scatter(hist, [addr], cnt, mask=lastm)
                    return 0
                lax.fori_loop(0, tc, _hc, 0)

                def _pr(i, carry):
                    h = hist[pl.ds(i * 16, 16)]
                    s = plsc.cumsum(h)
                    cum[pl.ds(i * 16, 16)] = s - h + _full16(carry)
                    return carry + jnp.sum(h)
                lax.fori_loop(0, tb, _pr, jnp.int32(0))

                def _b2r(i, nsl):
                    bidx = _iota16() + _full16(i * 16)
                    sz = hist[pl.ds(i * 16, 16)]
                    cm = cum[pl.ds(i * 16, 16)]
                    parent = _srl(bidx, 5)
                    sstart = plsc.load_gather(cum, [_sll(parent, 5)])
                    sbase = plsc.load_gather(sb_cur, [parent])
                    bb = sbase + cm - sstart
                    hast, _ = _first_target(bb, sz)
                    act = (sz > 1) & hast
                    ai = jnp.where(act, 1, 0).astype(jnp.int32)
                    pre = plsc.cumsum(ai) - ai + _full16(nsl)
                    smap[pl.ds(i * 16, 16)] = pre
                    plsc.store_scatter(sb_new, [jnp.where(act, pre, 0)], bb,
                                       mask=act)
                    return nsl + jnp.sum(ai)
                nslots = lax.fori_loop(0, tb, _b2r, jnp.int32(0))

                def _pp(i, off):
                    valid = _iota16() + _full16(i * 16) < _full16(ncand)
                    vv = vcand[pl.ds(i * 16, 16)]
                    iv = icand[pl.ds(i * 16, 16)]
                    sv = scand[pl.ds(i * 16, 16)]
                    dig = _srl(vv, sh_v) & 31
                    addr = jnp.where(valid, _sll(sv, 5) + dig, 0)
                    sz = plsc.load_gather(hist, [addr])
                    cm = plsc.load_gather(cum, [addr])
                    parent = _srl(addr, 5)
                    sstart = plsc.load_gather(cum, [_sll(parent, 5)])
                    sbase = plsc.load_gather(sb_cur, [parent])
                    b0 = sbase + cm - sstart
                    hast, _ = _first_target(b0, sz)
                    res1 = valid & hast & (sz == 1)
                    plsc.store_scatter(outb, [_srl(b0, 7)], iv, mask=res1)
                    alive = valid & hast & (sz > 1)
                    s_new = plsc.load_gather(smap, [addr])
                    offc = jnp.minimum(off, _CAP - 16)
                    am = jnp.where(alive, 1, 0).astype(jnp.int32)
                    plsc.store_compressed(vcand.at[pl.ds(offc, 16)], vv, mask=alive)
                    plsc.store_compressed(icand.at[pl.ds(offc, 16)], iv, mask=alive)
                    plsc.store_compressed(scand.at[pl.ds(offc, 16)], s_new,
                                          mask=alive)
                    return off + jnp.sum(am)
                ncand = lax.fori_loop(0, tc, _pp, jnp.int32(0))
                sb_cur, sb_new = sb_new, sb_cur

            def _zt(i, _):
                hist[pl.ds(i * 16, 16)] = zeros
                return 0
            lax.fori_loop(0, _NSLOT // 16, _zt, 0)

            def _tp(i, _):
                valid = _iota16() + _full16(i * 16) < _full16(ncand)
                sv = scand[pl.ds(i * 16, 16)]
                iv = icand[pl.ds(i * 16, 16)]
                addr = jnp.where(valid, sv, 0)
                cnt, lastm = plsc.scan_count(addr, mask=valid)
                prev = plsc.load_gather(hist, [addr])
                w = prev + cnt - 1
                r = plsc.load_gather(sb_cur, [addr]) + w
                emit = valid & ((r & 127) == 0)
                plsc.store_scatter(outb, [_srl(r, 7)], iv, mask=emit)
                plsc.addupdate_scatter(hist, [addr], cnt, mask=lastm)
                return 0
            lax.fori_loop(0, (ncand + 15) // 16, _tp, 0)

            pltpu.sync_copy(outb, out_hbm.at[row])

    return k


@functools.lru_cache(maxsize=None)
def _make_sc_gather(V, D, B):
    info = plsc.get_sparse_core_info()
    NC, NS = info.num_cores, info.num_subcores
    NW = NC * NS
    assert D % info.num_lanes == 0 and B % (8 * NW) == 0
    b_per_w = B // NW
    mesh = plsc.VectorSubcoreMesh(core_axis_name="c", subcore_axis_name="s")

    @functools.partial(
        pl.kernel,
        mesh=mesh,
        out_type=jax.ShapeDtypeStruct((B, D), jnp.float32),
        scratch_types=[
            pltpu.VMEM((b_per_w,), jnp.int32),
            pltpu.VMEM((b_per_w, D), jnp.float32),
            pltpu.SemaphoreType.DMA,
        ],
    )
    def k(table_hbm, idx_hbm, out_hbm, idx_v, rows_v, sem):
        wid = lax.axis_index("s") * NC + lax.axis_index("c")
        base = wid * b_per_w
        pltpu.sync_copy(idx_hbm.at[pl.ds(base, b_per_w)], idx_v)
        pltpu.async_copy(table_hbm.at[idx_v], rows_v, sem).wait()
        pltpu.sync_copy(rows_v, out_hbm.at[pl.ds(base, b_per_w)])

    return k


def kernel(features):
    B = features.shape[0]
    C = features.shape[-1]
    pts = features.reshape(B, -1, C)
    N = pts.shape[1]
    centroid = jnp.mean(pts, axis=1, keepdims=True)
    distances = jnp.linalg.norm(pts - centroid, axis=2)

    u = lax.bitcast_convert_type(distances, jnp.int32)
    umin = jnp.min(u, axis=1)
    rng = jnp.max(u, axis=1) - umin
    shift = jnp.maximum((32 - lax.clz(rng)) - 14, 0)
    prm = jnp.broadcast_to(
        jnp.stack([umin, shift], axis=1)[:, :, None], (B, 2, 16))

    sampled_indices = _make_select(B, N)(u, prm)

    flat_idx = (sampled_indices
                + (jnp.arange(B, dtype=jnp.int32) * N)[:, None]).reshape(-1)
    table = pts.reshape(B * N // 2, 2 * C)
    rows = _make_sc_gather(B * N // 2, 2 * C, B * _NUM_POINTS)(
        table, flat_idx >> 1)
    gathered = jnp.where((flat_idx & 1)[:, None] == 0, rows[:, :C], rows[:, C:])
    sampled_points = gathered.reshape(B, _NUM_POINTS, C)
    return (sampled_points, sampled_indices)
